# baseline (speedup 1.0000x reference)
import functools

import jax
import jax.numpy as jnp
from jax import lax
from jax.experimental import pallas as pl
from jax.experimental.pallas import tpu as pltpu
from jax.experimental.pallas import tpu_sc as plsc

HIDDEN = 1024
N_EXPERTS = 16
TOP_K = 8
MOE_INTER = 1024
SHARED_INTER = 2048
T_TOKENS = 2048

BLK = 256
G_BLOCKS = 79
N_CAP = G_BLOCKS * BLK
N_ASSIGN = T_TOKENS * TOP_K

_NC = 2
_NS = 16
_NW = _NC * _NS
TPT = T_TOKENS // _NW


def _router_meta_body(x_ref, gw_ref, w_ref, slot_ref, eg_ref):
    x = x_ref[...]
    gw = gw_ref[...]
    logits = lax.dot_general(x, gw, (((1,), (1,)), ((), ())),
                             preferred_element_type=jnp.float32)
    m = jnp.max(logits, axis=1, keepdims=True)
    p = jnp.exp(logits - m)
    s = p / jnp.sum(p, axis=1, keepdims=True)
    lane = lax.broadcasted_iota(jnp.int32, s.shape, 1)
    rank = jnp.zeros(s.shape, jnp.int32)
    for ep in range(N_EXPERTS):
        sp = s[:, ep:ep + 1]
        rank = rank + (sp > s).astype(jnp.int32)
        rank = rank + ((sp == s) & (ep < lane)).astype(jnp.int32)
    mask = rank < TOP_K
    w = jnp.where(mask, s, 0.0)
    w = w / jnp.sum(w, axis=1, keepdims=True)
    w_ref[...] = w

    maskf = mask.astype(jnp.float32)
    ti = lax.broadcasted_iota(jnp.int32, (T_TOKENS, T_TOKENS), 0)
    tj = lax.broadcasted_iota(jnp.int32, (T_TOKENS, T_TOKENS), 1)
    tril = (tj <= ti).astype(jnp.float32)
    cum = lax.dot_general(tril, maskf, (((1,), (0,)), ((), ())),
                          preferred_element_type=jnp.float32)
    cnt = cum[T_TOKENS - 1:T_TOKENS, :]
    nblk = (cnt.astype(jnp.int32) + (BLK - 1)) // BLK
    ei = lax.broadcasted_iota(jnp.int32, (N_EXPERTS, N_EXPERTS), 0)
    ej = lax.broadcasted_iota(jnp.int32, (N_EXPERTS, N_EXPERTS), 1)
    triu_strict = (ei < ej).astype(jnp.float32)
    offs_blk = lax.dot_general(nblk.astype(jnp.float32), triu_strict,
                               (((1,), (0,)), ((), ())),
                               preferred_element_type=jnp.float32)
    offs_blk = offs_blk.astype(jnp.int32)
    slot = offs_blk * BLK + cum.astype(jnp.int32) - 1
    eu = (ei <= ej).astype(jnp.float32)
    kcum = lax.dot_general(maskf, eu, (((1,), (0,)), ((), ())),
                           preferred_element_type=jnp.float32)
    kpos = kcum.astype(jnp.int32) - 1
    packed = slot + (kpos << 20)
    slot_ref[...] = jnp.where(mask, packed, 0)

    gi = lax.broadcasted_iota(jnp.int32, (256, N_EXPERTS), 0)
    le = (offs_blk <= gi).astype(jnp.int32)
    egv = jnp.sum(le, axis=1, keepdims=True) - 1
    egv = jnp.clip(egv, 0, N_EXPERTS - 1)
    eg_ref[...] = jnp.broadcast_to(egv, (256, 128))


def _shared_body(x_ref, gu_ref, dw_ref, out_ref):
    x = x_ref[...]
    gu = lax.dot_general(x, gu_ref[...], (((1,), (1,)), ((), ())),
                         preferred_element_type=jnp.float32)
    g = gu[:, :SHARED_INTER]
    u = gu[:, SHARED_INTER:]
    a = g * jax.nn.sigmoid(g) * u
    out_ref[...] = lax.dot_general(a, dw_ref[...], (((1,), (1,)), ((), ())),
                                   preferred_element_type=jnp.float32)


def _gmm_body(eg_sref, xs_ref, w1_ref, w2_ref, ys_ref):
    del eg_sref
    h = lax.dot_general(xs_ref[...], w1_ref[0], (((1,), (1,)), ((), ())),
                        preferred_element_type=jnp.float32)
    h = h * jax.nn.sigmoid(h)
    ys_ref[...] = lax.dot_general(h, w2_ref[0], (((1,), (1,)), ((), ())),
                                  preferred_element_type=jnp.float32)


def _sc_mesh():
    return plsc.VectorSubcoreMesh(core_axis_name="c", subcore_axis_name="s",
                                  num_cores=_NC, num_subcores=_NS)


def _make_dispatch():
    il16 = lambda: lax.broadcasted_iota(jnp.int32, (16,), 0)

    @functools.partial(
        pl.kernel,
        mesh=_sc_mesh(),
        compiler_params=pltpu.CompilerParams(needs_layout_passes=False),
        out_type=[
            jax.ShapeDtypeStruct((N_CAP, HIDDEN), jnp.float32),
            jax.ShapeDtypeStruct((N_ASSIGN,), jnp.int32),
            jax.ShapeDtypeStruct((N_ASSIGN,), jnp.float32),
        ],
        scratch_types=[
            pltpu.VMEM((TPT, HIDDEN), jnp.float32),
            pltpu.VMEM((TPT * N_EXPERTS,), jnp.int32),
            pltpu.VMEM((TPT * N_EXPERTS,), jnp.float32),
            pltpu.VMEM((8 * TPT + 16,), jnp.int32),
            pltpu.VMEM((TPT * 8 + 16,), jnp.int32),
            pltpu.VMEM((TPT * 8 + 16,), jnp.float32),
            pltpu.SemaphoreType.DMA,
        ],
    )
    def dispatch(x_hbm, slot_hbm, w_hbm, xs_hbm, s8_hbm, w8_hbm,
                 rows_v, st_v, wt_v, idx8_v, s8_v, w8_v, sem):
        wid = lax.axis_index("s") * _NC + lax.axis_index("c")
        base = wid * TPT
        pltpu.sync_copy(x_hbm.at[pl.ds(base, TPT)], rows_v)
        pltpu.sync_copy(slot_hbm.at[pl.ds(base * N_EXPERTS, TPT * N_EXPERTS)],
                        st_v)
        pltpu.sync_copy(w_hbm.at[pl.ds(base * N_EXPERTS, TPT * N_EXPERTS)],
                        wt_v)

        def tok_body(i, carry):
            packed = st_v[pl.ds(i * N_EXPERTS, 16)]
            wc = wt_v[pl.ds(i * N_EXPERTS, 16)]
            m = wc > 0.0
            kpos = lax.shift_right_logical(packed, 20)
            slotv = packed & 0xFFFFF
            lanes = il16()
            plsc.store_scatter(s8_v,
                               [jnp.where(m, i * 8 + kpos, TPT * 8 + lanes)],
                               slotv)
            plsc.store_scatter(w8_v,
                               [jnp.where(m, i * 8 + kpos, TPT * 8 + lanes)],
                               wc)
            ti = jnp.full((16,), i, jnp.int32)
            plsc.store_scatter(idx8_v,
                               [jnp.where(m, kpos * TPT + ti,
                                          8 * TPT + lanes)],
                               slotv)
            return carry

        lax.fori_loop(0, TPT, tok_body, 0)

        pltpu.sync_copy(s8_v.at[pl.ds(0, TPT * 8)],
                        s8_hbm.at[pl.ds(base * 8, TPT * 8)])
        pltpu.sync_copy(w8_v.at[pl.ds(0, TPT * 8)],
                        w8_hbm.at[pl.ds(base * 8, TPT * 8)])

        copies = []
        for k in range(8):
            for c in range(TPT // 16):
                idxvec = idx8_v[pl.ds(k * TPT + c * 16, 16)]
                copies.append(
                    pltpu.async_copy(rows_v.at[pl.ds(c * 16, 16)],
                                     xs_hbm.at[idxvec], sem))
        for cp in copies:
            cp.wait()

    return dispatch


_CH = 4


def _make_combine():
    @functools.partial(
        pl.kernel,
        mesh=_sc_mesh(),
        compiler_params=pltpu.CompilerParams(needs_layout_passes=False),
        out_type=jax.ShapeDtypeStruct((T_TOKENS, HIDDEN), jnp.float32),
        scratch_types=[
            pltpu.VMEM((TPT * 8,), jnp.int32),
            pltpu.VMEM((TPT * 8 + 16,), jnp.float32),
            pltpu.VMEM((2, _CH * 8, HIDDEN), jnp.float32),
            pltpu.VMEM((2, _CH, HIDDEN), jnp.float32),
            pltpu.VMEM((_CH, HIDDEN), jnp.float32),
            pltpu.SemaphoreType.DMA,
            pltpu.SemaphoreType.DMA,
        ],
    )
    def combine(ys_hbm, s8_hbm, w8_hbm, sh_hbm, out_hbm,
                idx_v, wv_v, rows_v, out_v, sh_v, sem0, sem1):
        wid = lax.axis_index("s") * _NC + lax.axis_index("c")
        base = wid * TPT
        pltpu.sync_copy(s8_hbm.at[pl.ds(base * 8, TPT * 8)], idx_v)
        pltpu.sync_copy(w8_hbm.at[pl.ds(base * 8, TPT * 8)],
                        wv_v.at[pl.ds(0, TPT * 8)])
        il16 = lax.broadcasted_iota(jnp.int32, (16,), 0)
        sems = [sem0, sem1]
        nch = TPT // _CH

        def fire(c, b):
            iv0 = idx_v[pl.ds(c * _CH * 8, 16)]
            iv1 = idx_v[pl.ds(c * _CH * 8 + 16, 16)]
            pltpu.async_copy(ys_hbm.at[iv0],
                             rows_v.at[b, pl.ds(0, 16)], sems[b])
            pltpu.async_copy(ys_hbm.at[iv1],
                             rows_v.at[b, pl.ds(16, 16)], sems[b])

        def drain(b):
            pltpu.make_async_copy(ys_hbm.at[pl.ds(0, _CH * 8)],
                                  rows_v.at[b], sems[b]).wait()

        fire(0, 0)
        fire(1, 1)

        def body2(j, carry):
            for b in range(2):
                c = j * 2 + b
                drain(b)
                pltpu.sync_copy(sh_hbm.at[pl.ds(base + c * _CH, _CH)], sh_v)
                for i in range(_CH):
                    w16 = wv_v[pl.ds(c * _CH * 8 + i * 8, 16)]
                    wb = [jnp.broadcast_to(
                            jnp.sum(jnp.where(il16 == k, w16, 0.0)), (16,))
                          for k in range(8)]

                    def vbody(v, carry2, _i=i, _wb=wb, _b=b):
                        for u in range(4):
                            sl = pl.ds((v * 4 + u) * 16, 16)
                            acc = sh_v[_i, sl] + _wb[0] * rows_v[_b, _i * 8,
                                                                 sl]
                            for k in range(1, 8):
                                acc = acc + _wb[k] * rows_v[_b, _i * 8 + k,
                                                            sl]
                            out_v[_b, _i, sl] = acc
                        return carry2

                    lax.fori_loop(0, HIDDEN // 64, vbody, 0)
                pltpu.sync_copy(out_v.at[b],
                                out_hbm.at[pl.ds(base + c * _CH, _CH)])

                @pl.when(c + 2 < nch)
                def _():
                    fire(c + 2, b)
            return carry

        lax.fori_loop(0, nch // 2, body2, 0)

    return combine


def kernel(hidden_states, gate_w, experts_w1, experts_w2,
           shared_gate_up_w, shared_down_w):
    orig_shape = hidden_states.shape
    x = hidden_states.reshape(-1, orig_shape[-1])
    T = x.shape[0]

    w_comb, slot, eg_pad = pl.pallas_call(
        _router_meta_body,
        out_shape=[
            jax.ShapeDtypeStruct((T, N_EXPERTS), jnp.float32),
            jax.ShapeDtypeStruct((T, N_EXPERTS), jnp.int32),
            jax.ShapeDtypeStruct((256, 128), jnp.int32),
        ],
    )(x, gate_w)
    eg_arr = eg_pad[:G_BLOCKS, 0]

    SB = 256
    shared_out = pl.pallas_call(
        _shared_body,
        grid=(T // SB,),
        in_specs=[
            pl.BlockSpec((SB, HIDDEN), lambda t: (t, 0)),
            pl.BlockSpec((2 * SHARED_INTER, HIDDEN), lambda t: (0, 0)),
            pl.BlockSpec((HIDDEN, SHARED_INTER), lambda t: (0, 0)),
        ],
        out_specs=pl.BlockSpec((SB, HIDDEN), lambda t: (t, 0)),
        out_shape=jax.ShapeDtypeStruct((T, HIDDEN), jnp.float32),
    )(x, shared_gate_up_w, shared_down_w)

    xs, slots8, w8 = _make_dispatch()(x, slot.reshape(-1),
                                      w_comb.reshape(-1))

    ys = pl.pallas_call(
        _gmm_body,
        grid_spec=pltpu.PrefetchScalarGridSpec(
            num_scalar_prefetch=1,
            grid=(G_BLOCKS,),
            in_specs=[
                pl.BlockSpec((BLK, HIDDEN), lambda g, eg: (g, 0)),
                pl.BlockSpec((1, MOE_INTER, HIDDEN),
                             lambda g, eg: (eg[g], 0, 0)),
                pl.BlockSpec((1, HIDDEN, MOE_INTER),
                             lambda g, eg: (eg[g], 0, 0)),
            ],
            out_specs=pl.BlockSpec((BLK, HIDDEN), lambda g, eg: (g, 0)),
        ),
        out_shape=jax.ShapeDtypeStruct((N_CAP, HIDDEN), jnp.float32),
        compiler_params=pltpu.CompilerParams(
            dimension_semantics=("arbitrary",),
        ),
    )(eg_arr, xs, experts_w1, experts_w2)

    out = _make_combine()(ys, slots8, w8, shared_out)

    return out.reshape(orig_shape)


# speedup vs baseline: 1.0740x; 1.0740x over previous
import functools

import jax
import jax.numpy as jnp
from jax import lax
from jax.experimental import pallas as pl
from jax.experimental.pallas import tpu as pltpu
from jax.experimental.pallas import tpu_sc as plsc

HIDDEN = 1024
N_EXPERTS = 16
TOP_K = 8
MOE_INTER = 1024
SHARED_INTER = 2048
T_TOKENS = 2048

BLK = 256
G_BLOCKS = 79
N_CAP = G_BLOCKS * BLK
N_ASSIGN = T_TOKENS * TOP_K

_NC = 2
_NS = 16
_NW = _NC * _NS
TPT = T_TOKENS // _NW


def _router_meta_body(x_ref, gw_ref, w_ref, slot_ref, eg_ref):
    x = x_ref[...]
    gw = gw_ref[...]
    logits = lax.dot_general(x, gw, (((1,), (1,)), ((), ())),
                             preferred_element_type=jnp.float32)
    m = jnp.max(logits, axis=1, keepdims=True)
    p = jnp.exp(logits - m)
    s = p / jnp.sum(p, axis=1, keepdims=True)
    lane = lax.broadcasted_iota(jnp.int32, s.shape, 1)
    rank = jnp.zeros(s.shape, jnp.int32)
    for ep in range(N_EXPERTS):
        sp = s[:, ep:ep + 1]
        rank = rank + (sp > s).astype(jnp.int32)
        rank = rank + ((sp == s) & (ep < lane)).astype(jnp.int32)
    mask = rank < TOP_K
    w = jnp.where(mask, s, 0.0)
    w = w / jnp.sum(w, axis=1, keepdims=True)
    w_ref[...] = w

    maskf = mask.astype(jnp.float32)
    ti = lax.broadcasted_iota(jnp.int32, (T_TOKENS, T_TOKENS), 0)
    tj = lax.broadcasted_iota(jnp.int32, (T_TOKENS, T_TOKENS), 1)
    tril = (tj <= ti).astype(jnp.float32)
    cum = lax.dot_general(tril, maskf, (((1,), (0,)), ((), ())),
                          preferred_element_type=jnp.float32)
    cnt = cum[T_TOKENS - 1:T_TOKENS, :]
    nblk = (cnt.astype(jnp.int32) + (BLK - 1)) // BLK
    ei = lax.broadcasted_iota(jnp.int32, (N_EXPERTS, N_EXPERTS), 0)
    ej = lax.broadcasted_iota(jnp.int32, (N_EXPERTS, N_EXPERTS), 1)
    triu_strict = (ei < ej).astype(jnp.float32)
    offs_blk = lax.dot_general(nblk.astype(jnp.float32), triu_strict,
                               (((1,), (0,)), ((), ())),
                               preferred_element_type=jnp.float32)
    offs_blk = offs_blk.astype(jnp.int32)
    slot = offs_blk * BLK + cum.astype(jnp.int32) - 1
    eu = (ei <= ej).astype(jnp.float32)
    kcum = lax.dot_general(maskf, eu, (((1,), (0,)), ((), ())),
                           preferred_element_type=jnp.float32)
    kpos = kcum.astype(jnp.int32) - 1
    packed = slot + (kpos << 20)
    slot_ref[...] = jnp.where(mask, packed, 0)

    gi = lax.broadcasted_iota(jnp.int32, (256, N_EXPERTS), 0)
    le = (offs_blk <= gi).astype(jnp.int32)
    egv = jnp.sum(le, axis=1, keepdims=True) - 1
    egv = jnp.clip(egv, 0, N_EXPERTS - 1)
    eg_ref[...] = jnp.broadcast_to(egv, (256, 128))


def _shared_body(x_ref, gu_ref, dw_ref, out_ref):
    x = x_ref[...]
    gu = lax.dot_general(x, gu_ref[...], (((1,), (1,)), ((), ())),
                         preferred_element_type=jnp.float32)
    g = gu[:, :SHARED_INTER]
    u = gu[:, SHARED_INTER:]
    a = g * jax.nn.sigmoid(g) * u
    out_ref[...] = lax.dot_general(a, dw_ref[...], (((1,), (1,)), ((), ())),
                                   preferred_element_type=jnp.float32)


def _gmm_body(eg_sref, xs_ref, w1_ref, w2_ref, ys_ref):
    del eg_sref
    h = lax.dot_general(xs_ref[...], w1_ref[0], (((1,), (1,)), ((), ())),
                        preferred_element_type=jnp.float32)
    h = h * jax.nn.sigmoid(h)
    ys_ref[...] = lax.dot_general(h, w2_ref[0], (((1,), (1,)), ((), ())),
                                  preferred_element_type=jnp.float32)


def _sc_mesh():
    return plsc.VectorSubcoreMesh(core_axis_name="c", subcore_axis_name="s",
                                  num_cores=_NC, num_subcores=_NS)


def _make_dispatch():
    il16 = lambda: lax.broadcasted_iota(jnp.int32, (16,), 0)

    @functools.partial(
        pl.kernel,
        mesh=_sc_mesh(),
        compiler_params=pltpu.CompilerParams(needs_layout_passes=False),
        out_type=[
            jax.ShapeDtypeStruct((N_CAP, HIDDEN), jnp.float32),
            jax.ShapeDtypeStruct((N_ASSIGN,), jnp.int32),
            jax.ShapeDtypeStruct((N_ASSIGN,), jnp.float32),
        ],
        scratch_types=[
            pltpu.VMEM((TPT, HIDDEN), jnp.float32),
            pltpu.VMEM((TPT * N_EXPERTS,), jnp.int32),
            pltpu.VMEM((TPT * N_EXPERTS,), jnp.float32),
            pltpu.VMEM((8 * TPT + 16,), jnp.int32),
            pltpu.VMEM((TPT * 8 + 16,), jnp.int32),
            pltpu.VMEM((TPT * 8 + 16,), jnp.float32),
            pltpu.SemaphoreType.DMA,
        ],
    )
    def dispatch(x_hbm, slot_hbm, w_hbm, xs_hbm, s8_hbm, w8_hbm,
                 rows_v, st_v, wt_v, idx8_v, s8_v, w8_v, sem):
        wid = lax.axis_index("s") * _NC + lax.axis_index("c")
        base = wid * TPT
        pltpu.sync_copy(x_hbm.at[pl.ds(base, TPT)], rows_v)
        pltpu.sync_copy(slot_hbm.at[pl.ds(base * N_EXPERTS, TPT * N_EXPERTS)],
                        st_v)
        pltpu.sync_copy(w_hbm.at[pl.ds(base * N_EXPERTS, TPT * N_EXPERTS)],
                        wt_v)

        def tok_body(i, carry):
            packed = st_v[pl.ds(i * N_EXPERTS, 16)]
            wc = wt_v[pl.ds(i * N_EXPERTS, 16)]
            m = wc > 0.0
            kpos = lax.shift_right_logical(packed, 20)
            slotv = packed & 0xFFFFF
            lanes = il16()
            plsc.store_scatter(s8_v,
                               [jnp.where(m, i * 8 + kpos, TPT * 8 + lanes)],
                               slotv)
            plsc.store_scatter(w8_v,
                               [jnp.where(m, i * 8 + kpos, TPT * 8 + lanes)],
                               wc)
            ti = jnp.full((16,), i, jnp.int32)
            plsc.store_scatter(idx8_v,
                               [jnp.where(m, kpos * TPT + ti,
                                          8 * TPT + lanes)],
                               slotv)
            return carry

        lax.fori_loop(0, TPT, tok_body, 0)

        pltpu.sync_copy(s8_v.at[pl.ds(0, TPT * 8)],
                        s8_hbm.at[pl.ds(base * 8, TPT * 8)])
        pltpu.sync_copy(w8_v.at[pl.ds(0, TPT * 8)],
                        w8_hbm.at[pl.ds(base * 8, TPT * 8)])

        copies = []
        for k in range(8):
            for c in range(TPT // 16):
                idxvec = idx8_v[pl.ds(k * TPT + c * 16, 16)]
                copies.append(
                    pltpu.async_copy(rows_v.at[pl.ds(c * 16, 16)],
                                     xs_hbm.at[idxvec], sem))
        for cp in copies:
            cp.wait()

    return dispatch


_CH = 4


def _make_combine():
    @functools.partial(
        pl.kernel,
        mesh=_sc_mesh(),
        compiler_params=pltpu.CompilerParams(needs_layout_passes=False),
        out_type=jax.ShapeDtypeStruct((T_TOKENS, HIDDEN), jnp.float32),
        scratch_types=[
            pltpu.VMEM((TPT * 8,), jnp.int32),
            pltpu.VMEM((TPT * 8 + 16,), jnp.float32),
            pltpu.VMEM((2, _CH * 8, HIDDEN), jnp.float32),
            pltpu.VMEM((2, _CH, HIDDEN), jnp.float32),
            pltpu.SemaphoreType.DMA,
            pltpu.SemaphoreType.DMA,
        ],
    )
    def combine(ys_hbm, s8_hbm, w8_hbm, out_hbm,
                idx_v, wv_v, rows_v, out_v, sem0, sem1):
        wid = lax.axis_index("s") * _NC + lax.axis_index("c")
        base = wid * TPT
        pltpu.sync_copy(s8_hbm.at[pl.ds(base * 8, TPT * 8)], idx_v)
        pltpu.sync_copy(w8_hbm.at[pl.ds(base * 8, TPT * 8)],
                        wv_v.at[pl.ds(0, TPT * 8)])
        il16 = lax.broadcasted_iota(jnp.int32, (16,), 0)
        sems = [sem0, sem1]
        nch = TPT // _CH

        def fire(c, b):
            iv0 = idx_v[pl.ds(c * _CH * 8, 16)]
            iv1 = idx_v[pl.ds(c * _CH * 8 + 16, 16)]
            pltpu.async_copy(ys_hbm.at[iv0],
                             rows_v.at[b, pl.ds(0, 16)], sems[b])
            pltpu.async_copy(ys_hbm.at[iv1],
                             rows_v.at[b, pl.ds(16, 16)], sems[b])

        def drain(b):
            pltpu.make_async_copy(ys_hbm.at[pl.ds(0, _CH * 8)],
                                  rows_v.at[b], sems[b]).wait()

        fire(0, 0)
        fire(1, 1)

        def body2(j, carry):
            for b in range(2):
                c = j * 2 + b
                drain(b)
                for i in range(_CH):
                    w16 = wv_v[pl.ds(c * _CH * 8 + i * 8, 16)]
                    wb = [jnp.broadcast_to(
                            jnp.sum(jnp.where(il16 == k, w16, 0.0)), (16,))
                          for k in range(8)]

                    def vbody(v, carry2, _i=i, _wb=wb, _b=b):
                        for u in range(4):
                            sl = pl.ds((v * 4 + u) * 16, 16)
                            acc = _wb[0] * rows_v[_b, _i * 8, sl]
                            for k in range(1, 8):
                                acc = acc + _wb[k] * rows_v[_b, _i * 8 + k,
                                                            sl]
                            out_v[_b, _i, sl] = acc
                        return carry2

                    lax.fori_loop(0, HIDDEN // 64, vbody, 0)
                pltpu.sync_copy(out_v.at[b],
                                out_hbm.at[pl.ds(base + c * _CH, _CH)])

                @pl.when(c + 2 < nch)
                def _():
                    fire(c + 2, b)
            return carry

        lax.fori_loop(0, nch // 2, body2, 0)

    return combine


def _add_body(a_ref, b_ref, out_ref):
    out_ref[...] = a_ref[...] + b_ref[...]


def kernel(hidden_states, gate_w, experts_w1, experts_w2,
           shared_gate_up_w, shared_down_w):
    orig_shape = hidden_states.shape
    x = hidden_states.reshape(-1, orig_shape[-1])
    T = x.shape[0]

    w_comb, slot, eg_pad = pl.pallas_call(
        _router_meta_body,
        out_shape=[
            jax.ShapeDtypeStruct((T, N_EXPERTS), jnp.float32),
            jax.ShapeDtypeStruct((T, N_EXPERTS), jnp.int32),
            jax.ShapeDtypeStruct((256, 128), jnp.int32),
        ],
    )(x, gate_w)
    eg_arr = eg_pad[:G_BLOCKS, 0]

    SB = 256
    shared_out = pl.pallas_call(
        _shared_body,
        grid=(T // SB,),
        in_specs=[
            pl.BlockSpec((SB, HIDDEN), lambda t: (t, 0)),
            pl.BlockSpec((2 * SHARED_INTER, HIDDEN), lambda t: (0, 0)),
            pl.BlockSpec((HIDDEN, SHARED_INTER), lambda t: (0, 0)),
        ],
        out_specs=pl.BlockSpec((SB, HIDDEN), lambda t: (t, 0)),
        out_shape=jax.ShapeDtypeStruct((T, HIDDEN), jnp.float32),
    )(x, shared_gate_up_w, shared_down_w)

    xs, slots8, w8 = _make_dispatch()(x, slot.reshape(-1),
                                      w_comb.reshape(-1))

    ys = pl.pallas_call(
        _gmm_body,
        grid_spec=pltpu.PrefetchScalarGridSpec(
            num_scalar_prefetch=1,
            grid=(G_BLOCKS,),
            in_specs=[
                pl.BlockSpec((BLK, HIDDEN), lambda g, eg: (g, 0)),
                pl.BlockSpec((1, MOE_INTER, HIDDEN),
                             lambda g, eg: (eg[g], 0, 0)),
                pl.BlockSpec((1, HIDDEN, MOE_INTER),
                             lambda g, eg: (eg[g], 0, 0)),
            ],
            out_specs=pl.BlockSpec((BLK, HIDDEN), lambda g, eg: (g, 0)),
        ),
        out_shape=jax.ShapeDtypeStruct((N_CAP, HIDDEN), jnp.float32),
        compiler_params=pltpu.CompilerParams(
            dimension_semantics=("arbitrary",),
        ),
    )(eg_arr, xs, experts_w1, experts_w2)

    routed = _make_combine()(ys, slots8, w8)

    AB = 256
    out = pl.pallas_call(
        _add_body,
        grid=(T // AB,),
        in_specs=[
            pl.BlockSpec((AB, HIDDEN), lambda t: (t, 0)),
            pl.BlockSpec((AB, HIDDEN), lambda t: (t, 0)),
        ],
        out_specs=pl.BlockSpec((AB, HIDDEN), lambda t: (t, 0)),
        out_shape=jax.ShapeDtypeStruct((T, HIDDEN), jnp.float32),
    )(routed, shared_out)

    return out.reshape(orig_shape)
